# baseline (speedup 1.0000x reference)
import functools

import jax
import jax.numpy as jnp
from jax import lax
from jax.experimental import pallas as pl
from jax.experimental.pallas import tpu as pltpu
from jax.experimental.pallas import tpu_sc as plsc

B = 16384
S = 77
D = 64
DP = 128
EOS = 49407
NC = 2
NS = 16
NW = NC * NS
ROWS_PER_W = B // NW
CB = 1
NCHUNK = ROWS_PER_W // CB
LANES = 16


def _emb_body(ids_hbm, table_hbm, pos_hbm, out_hbm, pos_v, idx_v, rows_v,
              sbuf_v, sem):
    wid = lax.axis_index("s") * NC + lax.axis_index("c")
    base_row = wid * ROWS_PER_W

    pltpu.sync_copy(pos_hbm, pos_v)

    def chunk_body(g, carry):
        row0 = base_row + g * CB
        pltpu.sync_copy(ids_hbm.at[pl.ds(row0, CB)], idx_v)
        for c in range(CB):
            pltpu.async_copy(table_hbm.at[idx_v.at[c]], rows_v.at[c], sem)
        for c in range(CB):
            pltpu.make_async_copy(table_hbm.at[idx_v.at[c]], rows_v.at[c],
                                  sem).wait()

        def s_body(s, carry2):
            for j in range(D // LANES):
                p = pos_v[s, pl.ds(j * LANES, LANES)]
                for c in range(CB):
                    sbuf_v[c, s, pl.ds(j * LANES, LANES)] = (
                        rows_v[c, s, pl.ds(j * LANES, LANES)] + p)
            return carry2

        lax.fori_loop(0, S, s_body, 0)
        pltpu.sync_copy(sbuf_v, out_hbm.at[pl.ds(row0, CB)])
        return carry

    lax.fori_loop(0, NCHUNK, chunk_body, 0)


_emb = functools.partial(
    pl.kernel,
    out_type=jax.ShapeDtypeStruct((B, S, D), jnp.float32),
    mesh=plsc.VectorSubcoreMesh(core_axis_name="c", subcore_axis_name="s"),
    scratch_types=[
        pltpu.VMEM((S, D), jnp.float32),
        pltpu.VMEM((CB, S), jnp.int32),
        pltpu.VMEM((CB, S, DP), jnp.float32),
        pltpu.VMEM((CB, S, D), jnp.float32),
        pltpu.SemaphoreType.DMA,
    ],
)(_emb_body)


def _mask_body(ids_ref, m_ref):
    m_ref[...] = ids_ref[...] == EOS


_mask = pl.pallas_call(
    _mask_body,
    out_shape=jax.ShapeDtypeStruct((B, S), jnp.bool_),
)


def kernel(input_ids, text_embedding, positional_embedding):
    ids = input_ids.astype(jnp.int32)
    table_p = jnp.pad(text_embedding, ((0, 0), (0, DP - D)))
    tokens = _emb(ids, table_p, positional_embedding)
    mask = _mask(ids)
    return tokens, mask


# speedup vs baseline: 1.7087x; 1.7087x over previous
import functools

import jax
import jax.numpy as jnp
from jax import lax
from jax.experimental import pallas as pl
from jax.experimental.pallas import tpu as pltpu
from jax.experimental.pallas import tpu_sc as plsc

B = 16384
S = 77
D = 64
DP = 128
EOS = 49407
NC = 2
NS = 16
NW = NC * NS
ROWS_PER_W = B // NW
CB = 1
NCHUNK = ROWS_PER_W // CB
NBUF = 4
LANES = 16


def _emb_body(ids_hbm, table_hbm, pos_hbm, out_hbm, pos_v,
              idx0, idx1, idx2, idx3, gb0, gb1, gb2, gb3,
              sb0, sb1, sb2, sb3,
              si0, si1, si2, si3, sg0, sg1, sg2, sg3, ss0, ss1, ss2, ss3):
    idx = [idx0, idx1, idx2, idx3]
    gb = [gb0, gb1, gb2, gb3]
    sb = [sb0, sb1, sb2, sb3]
    si = [si0, si1, si2, si3]
    sg = [sg0, sg1, sg2, sg3]
    ss = [ss0, ss1, ss2, ss3]

    wid = lax.axis_index("s") * NC + lax.axis_index("c")
    base_row = wid * ROWS_PER_W

    pltpu.sync_copy(pos_hbm, pos_v)

    def start_idx(b, g):
        row0 = base_row + g * CB
        pltpu.async_copy(ids_hbm.at[pl.ds(row0, CB)], idx[b], si[b])

    def wait_idx(b, g):
        row0 = base_row + g * CB
        pltpu.make_async_copy(ids_hbm.at[pl.ds(row0, CB)], idx[b], si[b]).wait()

    def start_gathers(b):
        for c in range(CB):
            pltpu.async_copy(table_hbm.at[idx[b].at[c]], gb[b].at[c], sg[b])

    def wait_gathers(b):
        for c in range(CB):
            pltpu.make_async_copy(table_hbm.at[idx[b].at[c]], gb[b].at[c],
                                  sg[b]).wait()

    def add_pos(b):
        def s_body(s, carry):
            for j in range(D // LANES):
                p = pos_v[s, pl.ds(j * LANES, LANES)]
                for c in range(CB):
                    sb[b][c, s, pl.ds(j * LANES, LANES)] = (
                        gb[b][c, s, pl.ds(j * LANES, LANES)] + p)
            return carry

        lax.fori_loop(0, S, s_body, 0)

    def start_store(b, g):
        row0 = base_row + g * CB
        pltpu.async_copy(sb[b], out_hbm.at[pl.ds(row0, CB)], ss[b])

    def wait_store(b, g):
        row0 = base_row + g * CB
        pltpu.make_async_copy(sb[b], out_hbm.at[pl.ds(row0, CB)], ss[b]).wait()

    for h in range(3):
        start_idx(h, h)
    for h in range(2):
        wait_idx(h, h)
        start_gathers(h)

    def outer_body(i, carry):
        for bb in range(NBUF):
            g = i * NBUF + bb
            b = bb
            wait_gathers(b)
            add_pos(b)
            start_store(b, g)
            bn = (bb + 2) % NBUF
            bi = (bb + 3) % NBUF

            @pl.when(g + 2 < NCHUNK)
            def _():
                @pl.when(g >= 2)
                def _():
                    wait_store(bn, g - 2)

                wait_idx(bn, g + 2)
                start_gathers(bn)

            @pl.when(g + 3 < NCHUNK)
            def _():
                start_idx(bi, g + 3)

        return carry

    lax.fori_loop(0, NCHUNK // NBUF, outer_body, 0)

    for k in range(NBUF):
        g = NCHUNK - NBUF + k
        wait_store(g % NBUF, g)


_scr_idx = [pltpu.VMEM((CB, S), jnp.int32) for _ in range(NBUF)]
_scr_gb = [pltpu.VMEM((CB, S, DP), jnp.float32) for _ in range(NBUF)]
_scr_sb = [pltpu.VMEM((CB, S, D), jnp.float32) for _ in range(NBUF)]
_scr_sem = [pltpu.SemaphoreType.DMA for _ in range(3 * NBUF)]

_emb = functools.partial(
    pl.kernel,
    out_type=jax.ShapeDtypeStruct((B, S, D), jnp.float32),
    mesh=plsc.VectorSubcoreMesh(core_axis_name="c", subcore_axis_name="s"),
    scratch_types=[pltpu.VMEM((S, D), jnp.float32)]
    + _scr_idx + _scr_gb + _scr_sb + _scr_sem,
)(_emb_body)


def _mask_body(ids_ref, m_ref):
    m_ref[...] = ids_ref[...] == EOS


_mask = pl.pallas_call(
    _mask_body,
    out_shape=jax.ShapeDtypeStruct((B, S), jnp.bool_),
)


def kernel(input_ids, text_embedding, positional_embedding):
    ids = input_ids.astype(jnp.int32)
    table_p = jnp.pad(text_embedding, ((0, 0), (0, DP - D)))
    tokens = _emb(ids, table_p, positional_embedding)
    mask = _mask(ids)
    return tokens, mask


# speedup vs baseline: 1.7113x; 1.0015x over previous
import functools

import jax
import jax.numpy as jnp
from jax import lax
from jax.experimental import pallas as pl
from jax.experimental.pallas import tpu as pltpu
from jax.experimental.pallas import tpu_sc as plsc

B = 16384
S = 77
D = 64
DP = 128
EOS = 49407
NC = 2
NS = 16
NW = NC * NS
ROWS_PER_W = B // NW
CB = 1
NCHUNK = ROWS_PER_W // CB
NBUF = 4
LANES = 16


def _emb_body(ids_hbm, table_hbm, pos_hbm, out_hbm, pos_sb,
              idx0, idx1, idx2, idx3, gb0, gb1, gb2, gb3,
              sb0, sb1, sb2, sb3,
              si0, si1, si2, si3, sg0, sg1, sg2, sg3, ss0, ss1, ss2, ss3,
              sp0, sp1, sp2, sp3):
    idx = [idx0, idx1, idx2, idx3]
    gb = [gb0, gb1, gb2, gb3]
    sb = [sb0, sb1, sb2, sb3]
    si = [si0, si1, si2, si3]
    sg = [sg0, sg1, sg2, sg3]
    ss = [ss0, ss1, ss2, ss3]
    sp = [sp0, sp1, sp2, sp3]

    wid = lax.axis_index("s") * NC + lax.axis_index("c")
    base_row = wid * ROWS_PER_W

    @pl.when(lax.axis_index("s") == 0)
    def _():
        pltpu.sync_copy(pos_hbm, pos_sb)

    plsc.subcore_barrier()

    def start_idx(b, g):
        row0 = base_row + g * CB
        pltpu.async_copy(ids_hbm.at[pl.ds(row0, CB)], idx[b], si[b])

    def wait_idx(b, g):
        row0 = base_row + g * CB
        pltpu.make_async_copy(ids_hbm.at[pl.ds(row0, CB)], idx[b], si[b]).wait()

    def start_gathers(b):
        for c in range(CB):
            pltpu.async_copy(table_hbm.at[idx[b].at[c]], gb[b].at[c], sg[b])

    def wait_gathers(b):
        for c in range(CB):
            pltpu.make_async_copy(table_hbm.at[idx[b].at[c]], gb[b].at[c],
                                  sg[b]).wait()

    def start_prefill(b):
        pltpu.async_copy(pos_sb, sb[b].at[0], sp[b])

    def wait_prefill(b):
        pltpu.make_async_copy(pos_sb, sb[b].at[0], sp[b]).wait()

    def add_pos(b):
        def s_body(s, carry):
            for c in range(CB):
                for j in range(D // LANES):
                    x = gb[b][c, s, pl.ds(j * LANES, LANES)]
                    plsc.addupdate(sb[b].at[c, s, pl.ds(j * LANES, LANES)], x)
            return carry

        lax.fori_loop(0, S, s_body, 0, unroll=7)

    def start_store(b, g):
        row0 = base_row + g * CB
        pltpu.async_copy(sb[b], out_hbm.at[pl.ds(row0, CB)], ss[b])

    def wait_store(b, g):
        row0 = base_row + g * CB
        pltpu.make_async_copy(sb[b], out_hbm.at[pl.ds(row0, CB)], ss[b]).wait()

    for h in range(3):
        start_idx(h, h)
    for h in range(2):
        wait_idx(h, h)
        start_gathers(h)
        start_prefill(h)

    def outer_body(i, carry):
        for bb in range(NBUF):
            g = i * NBUF + bb
            b = bb
            wait_gathers(b)
            wait_prefill(b)
            add_pos(b)
            start_store(b, g)
            bn = (bb + 2) % NBUF
            bi = (bb + 3) % NBUF

            @pl.when(g + 2 < NCHUNK)
            def _():
                @pl.when(g >= 2)
                def _():
                    wait_store(bn, g - 2)

                start_prefill(bn)
                wait_idx(bn, g + 2)
                start_gathers(bn)

            @pl.when(g + 3 < NCHUNK)
            def _():
                start_idx(bi, g + 3)

        return carry

    lax.fori_loop(0, NCHUNK // NBUF, outer_body, 0)

    for k in range(NBUF):
        g = NCHUNK - NBUF + k
        wait_store(g % NBUF, g)


_scr_idx = [pltpu.VMEM((CB, S), jnp.int32) for _ in range(NBUF)]
_scr_gb = [pltpu.VMEM((CB, S, DP), jnp.float32) for _ in range(NBUF)]
_scr_sb = [pltpu.VMEM((CB, S, D), jnp.float32) for _ in range(NBUF)]
_scr_sem = [pltpu.SemaphoreType.DMA for _ in range(4 * NBUF)]

_emb = functools.partial(
    pl.kernel,
    out_type=jax.ShapeDtypeStruct((B, S, D), jnp.float32),
    mesh=plsc.VectorSubcoreMesh(core_axis_name="c", subcore_axis_name="s"),
    scratch_types=[pltpu.VMEM_SHARED((S, D), jnp.float32)]
    + _scr_idx + _scr_gb + _scr_sb + _scr_sem,
)(_emb_body)


def _mask_body(ids_ref, m_ref):
    m_ref[...] = ids_ref[...] == EOS


_mask = pl.pallas_call(
    _mask_body,
    out_shape=jax.ShapeDtypeStruct((B, S), jnp.bool_),
)


def kernel(input_ids, text_embedding, positional_embedding):
    ids = input_ids.astype(jnp.int32)
    table_p = jnp.pad(text_embedding, ((0, 0), (0, DP - D)))
    tokens = _emb(ids, table_p, positional_embedding)
    mask = _mask(ids)
    return tokens, mask
